# baseline (speedup 1.0000x reference)
import functools

import jax
import jax.numpy as jnp
from jax import lax
from jax.experimental import pallas as pl
from jax.experimental.pallas import tpu as pltpu
from jax.experimental.pallas import tpu_sc as plsc

_F = 26
_V = 20
_VP = 32
_E = 64
_A = 20.0
_NT = _F * _VP
_NTP = 896


def _dg(a16, i16):
    return lax.gather(
        a16,
        i16[:, None],
        lax.GatherDimensionNumbers(
            offset_dims=(), collapsed_slice_dims=(0,), start_index_map=(0,)
        ),
        (1,),
        mode=lax.GatherScatterMode.PROMISE_IN_BOUNDS,
    )


def _tc_prep(primt_ref, dfltt_ref, cnt_ref, xt_ref, blendt_ref, xtp_ref):
    w = cnt_ref[...] / (cnt_ref[...] + _A)
    blendt_ref[...] = w * primt_ref[...] + (1.0 - w) * dfltt_ref[...]
    xtp_ref[...] = jnp.concatenate(
        [xt_ref[...], jnp.zeros((32 - _F, xt_ref.shape[1]), jnp.int32)], axis=0
    )


def kernel(X, emb_table, counts):
    B = X.shape[0]
    NBT = B // 128

    embt3 = jnp.transpose(emb_table, (1, 0)).reshape(_E, _F, _V + 1)
    primt = embt3[:, :, 1:]
    dfltt = jnp.broadcast_to(embt3[:, :, :1], (_E, _F, _V))
    NR = _E * _NTP // 128
    NCT = _NTP // 128
    primt = jnp.pad(primt, ((0, 0), (0, 0), (0, _VP - _V))).reshape(_E, _NT)
    dfltt = jnp.pad(dfltt, ((0, 0), (0, 0), (0, _VP - _V))).reshape(_E, _NT)
    primt = jnp.pad(primt, ((0, 0), (0, _NTP - _NT)))
    dfltt = jnp.pad(dfltt, ((0, 0), (0, _NTP - _NT)))
    primt = jnp.transpose(primt.reshape(_E, NCT, 128), (1, 0, 2)).reshape(NR, 128)
    dfltt = jnp.transpose(dfltt.reshape(_E, NCT, 128), (1, 0, 2)).reshape(NR, 128)
    cntp = jnp.pad(counts, ((0, 0), (0, _VP - _V))).reshape(_NT)
    cntp = jnp.pad(cntp, (0, _NTP - _NT)).astype(jnp.float32)
    cntp = jnp.broadcast_to(cntp.reshape(NCT, 1, 128), (NCT, _E, 128)).reshape(NR, 128)

    blendt, xtp = pl.pallas_call(
        _tc_prep,
        out_shape=(
            jax.ShapeDtypeStruct((NR, 128), jnp.float32),
            jax.ShapeDtypeStruct((32, B), jnp.int32),
        ),
    )(primt, dfltt, cntp, jnp.transpose(X, (1, 0)))

    x4 = xtp.reshape(4, 8, NBT, 128).transpose(0, 2, 1, 3)

    info = plsc.get_sparse_core_info()
    NC, NS = info.num_cores, info.num_subcores
    NW = NC * NS
    assert NW == NBT

    mesh = plsc.VectorSubcoreMesh(core_axis_name="c", subcore_axis_name="s")

    @functools.partial(
        pl.kernel,
        out_type=jax.ShapeDtypeStruct((_F, 8, NBT, 8, 128), jnp.float32),
        mesh=mesh,
        compiler_params=pltpu.CompilerParams(
            use_tc_tiling_on_sc=False, needs_layout_passes=False
        ),
        scratch_types=[
            pltpu.VMEM((_E * _NTP // 128, 128), jnp.float32),
            pltpu.VMEM((4, 8, 128), jnp.int32),
            pltpu.VMEM((8, 8, 128), jnp.float32),
            pltpu.VMEM((8, 8, 128), jnp.float32),
            pltpu.SemaphoreType.DMA,
            pltpu.SemaphoreType.DMA,
            pltpu.SemaphoreType.DMA,
            pltpu.SemaphoreType.DMA,
        ],
    )
    def sc_fill(
        x_hbm, tbl_hbm, out_hbm, tbl_v, idx_v, obuf0, obuf1, sem0, sem1, tsa, tsb
    ):
        wid = lax.axis_index("s") * NC + lax.axis_index("c")
        cpy_a = pltpu.async_copy(tbl_hbm.at[pl.ds(0, 128)], tbl_v.at[pl.ds(0, 128)], tsa)
        pltpu.async_copy(
            tbl_hbm.at[pl.ds(128, 320)], tbl_v.at[pl.ds(128, 320)], tsb
        )
        pltpu.sync_copy(x_hbm.at[:, wid], idx_v)
        cpy_a.wait()

        def out_slice(f):
            return out_hbm.at[f, :, wid]

        def chunk(j, obuf, sem):
            ct64 = (j // 4) * _E
            ci = (j % 4) * _VP

            @pl.when(j == 8)
            def _():
                pltpu.make_async_copy(
                    tbl_hbm.at[pl.ds(128, 320)], tbl_v.at[pl.ds(128, 320)], tsb
                ).wait()

            xs, ms = [], []
            for c in range(8):
                x16 = idx_v[j // 8, j % 8, pl.ds(c * 16, 16)]
                xs.append(x16 & 15)
                ms.append(x16 < 16)
            lo = tbl_v[ct64, pl.ds(ci, 16)]
            hi = tbl_v[ct64, pl.ds(ci + 16, 16)]

            @pl.when(j >= 2)
            def _():
                pltpu.make_async_copy(obuf, out_slice(j - 2), sem).wait()
            for e in range(_E):
                if e + 1 < _E:
                    r = ct64 + e + 1
                    lo_n = tbl_v[r, pl.ds(ci, 16)]
                    hi_n = tbl_v[r, pl.ds(ci + 16, 16)]
                for c in range(8):
                    obuf[e // 8, e % 8, pl.ds(c * 16, 16)] = jnp.where(
                        ms[c], _dg(lo, xs[c]), _dg(hi, xs[c])
                    )
                if e + 1 < _E:
                    lo, hi = lo_n, hi_n
            pltpu.async_copy(obuf, out_slice(j), sem)

        def body(i, carry):
            chunk(2 * i, obuf0, sem0)
            chunk(2 * i + 1, obuf1, sem1)
            return carry

        lax.fori_loop(0, _F // 2, body, 0)
        pltpu.make_async_copy(obuf0, out_slice(_F - 2), sem0).wait()
        pltpu.make_async_copy(obuf1, out_slice(_F - 1), sem1).wait()

    q = sc_fill(x4, blendt)
    return q.transpose((2, 4, 0, 1, 3)).reshape(B, _F, _E)


# speedup vs baseline: 1.0245x; 1.0245x over previous
import functools

import jax
import jax.numpy as jnp
from jax import lax
from jax.experimental import pallas as pl
from jax.experimental.pallas import tpu as pltpu
from jax.experimental.pallas import tpu_sc as plsc

_F = 26
_V = 20
_VP = 32
_E = 64
_A = 20.0
_NT = _F * _VP
_NTP = 896


def _dg(a16, i16):
    return lax.gather(
        a16,
        i16[:, None],
        lax.GatherDimensionNumbers(
            offset_dims=(), collapsed_slice_dims=(0,), start_index_map=(0,)
        ),
        (1,),
        mode=lax.GatherScatterMode.PROMISE_IN_BOUNDS,
    )


def _tc_prep(primt_ref, dfltt_ref, cnt_ref, xt_ref, blendt_ref, xtp_ref):
    w = cnt_ref[...] / (cnt_ref[...] + _A)
    blendt_ref[...] = w * primt_ref[...] + (1.0 - w) * dfltt_ref[...]
    xtp_ref[...] = jnp.concatenate(
        [xt_ref[...], jnp.zeros((32 - _F, xt_ref.shape[1]), jnp.int32)], axis=0
    )


def kernel(X, emb_table, counts):
    B = X.shape[0]
    NBT = B // 128

    embt3 = jnp.transpose(emb_table, (1, 0)).reshape(_E, _F, _V + 1)
    primt = embt3[:, :, 1:]
    dfltt = jnp.broadcast_to(embt3[:, :, :1], (_E, _F, _V))
    NR = _E * _NTP // 128
    primt = jnp.pad(primt, ((0, 0), (0, 0), (0, _VP - _V))).reshape(_E, _NT)
    dfltt = jnp.pad(dfltt, ((0, 0), (0, 0), (0, _VP - _V))).reshape(_E, _NT)
    primt = jnp.pad(primt, ((0, 0), (0, _NTP - _NT))).reshape(NR, 128)
    dfltt = jnp.pad(dfltt, ((0, 0), (0, _NTP - _NT))).reshape(NR, 128)
    cntp = jnp.pad(counts, ((0, 0), (0, _VP - _V))).reshape(_NT)
    cntp = jnp.pad(cntp, (0, _NTP - _NT)).astype(jnp.float32)
    cntp = jnp.broadcast_to(cntp.reshape(1, _NTP // 128, 128), (_E, _NTP // 128, 128)).reshape(NR, 128)

    blendt, xtp = pl.pallas_call(
        _tc_prep,
        out_shape=(
            jax.ShapeDtypeStruct((NR, 128), jnp.float32),
            jax.ShapeDtypeStruct((32, B), jnp.int32),
        ),
    )(primt, dfltt, cntp, jnp.transpose(X, (1, 0)))

    x4 = xtp.reshape(4, 8, NBT, 128).transpose(0, 2, 1, 3)

    info = plsc.get_sparse_core_info()
    NC, NS = info.num_cores, info.num_subcores
    NW = NC * NS
    assert NW == NBT

    mesh = plsc.VectorSubcoreMesh(core_axis_name="c", subcore_axis_name="s")

    @functools.partial(
        pl.kernel,
        out_type=jax.ShapeDtypeStruct((_F, 8, NBT, 8, 128), jnp.float32),
        mesh=mesh,
        compiler_params=pltpu.CompilerParams(
            use_tc_tiling_on_sc=False, needs_layout_passes=False
        ),
        scratch_types=[
            pltpu.VMEM((_E * _NTP // 128, 128), jnp.float32),
            pltpu.VMEM((4, 8, 128), jnp.int32),
            pltpu.VMEM((8, 8, 128), jnp.float32),
            pltpu.VMEM((8, 8, 128), jnp.float32),
            pltpu.SemaphoreType.DMA,
            pltpu.SemaphoreType.DMA,
        ],
    )
    def sc_fill(x_hbm, tbl_hbm, out_hbm, tbl_v, idx_v, obuf0, obuf1, sem0, sem1):
        wid = lax.axis_index("s") * NC + lax.axis_index("c")
        pltpu.sync_copy(tbl_hbm, tbl_v)
        pltpu.sync_copy(x_hbm.at[:, wid], idx_v)

        def out_slice(f):
            return out_hbm.at[f, :, wid]

        def chunk(j, obuf, sem):
            ct = j // 4
            ci = (j % 4) * _VP

            xs, ms = [], []
            for c in range(8):
                x16 = idx_v[j // 8, j % 8, pl.ds(c * 16, 16)]
                xs.append(x16 & 15)
                ms.append(x16 < 16)
            lo = tbl_v[ct, pl.ds(ci, 16)]
            hi = tbl_v[ct, pl.ds(ci + 16, 16)]

            @pl.when(j >= 2)
            def _():
                pltpu.make_async_copy(obuf, out_slice(j - 2), sem).wait()
            for e in range(_E):
                if e + 1 < _E:
                    r = 7 * (e + 1) + ct
                    lo_n = tbl_v[r, pl.ds(ci, 16)]
                    hi_n = tbl_v[r, pl.ds(ci + 16, 16)]
                for c in range(8):
                    obuf[e // 8, e % 8, pl.ds(c * 16, 16)] = jnp.where(
                        ms[c], _dg(lo, xs[c]), _dg(hi, xs[c])
                    )
                if e + 1 < _E:
                    lo, hi = lo_n, hi_n
            pltpu.async_copy(obuf, out_slice(j), sem)

        def body(i, carry):
            chunk(2 * i, obuf0, sem0)
            chunk(2 * i + 1, obuf1, sem1)
            return carry

        lax.fori_loop(0, _F // 2, body, 0)
        pltpu.make_async_copy(obuf0, out_slice(_F - 2), sem0).wait()
        pltpu.make_async_copy(obuf1, out_slice(_F - 1), sem1).wait()

    q = sc_fill(x4, blendt)
    return q.transpose((2, 4, 0, 1, 3)).reshape(B, _F, _E)


# speedup vs baseline: 1.0263x; 1.0018x over previous
import functools

import jax
import jax.numpy as jnp
from jax import lax
from jax.experimental import pallas as pl
from jax.experimental.pallas import tpu as pltpu
from jax.experimental.pallas import tpu_sc as plsc

_F = 26
_V = 20
_VP = 32
_E = 64
_A = 20.0
_NT = _F * _VP
_NTP = 896


def _dg(a16, i16):
    return lax.gather(
        a16,
        i16[:, None],
        lax.GatherDimensionNumbers(
            offset_dims=(), collapsed_slice_dims=(0,), start_index_map=(0,)
        ),
        (1,),
        mode=lax.GatherScatterMode.PROMISE_IN_BOUNDS,
    )


def _tc_prep(primt_ref, dfltt_ref, cnt_ref, xt_ref, blendt_ref, xtp_ref):
    w = cnt_ref[...] / (cnt_ref[...] + _A)
    blendt_ref[...] = w * primt_ref[...] + (1.0 - w) * dfltt_ref[...]
    xtp_ref[...] = jnp.concatenate(
        [xt_ref[...], jnp.zeros((32 - _F, xt_ref.shape[1]), jnp.int32)], axis=0
    )


def kernel(X, emb_table, counts):
    B = X.shape[0]
    NBT = B // 128

    embt3 = jnp.transpose(emb_table, (1, 0)).reshape(_E, _F, _V + 1)
    primt = embt3[:, :, 1:]
    dfltt = jnp.broadcast_to(embt3[:, :, :1], (_E, _F, _V))
    NR = _E * _NTP // 128
    primt = jnp.pad(primt, ((0, 0), (0, 0), (0, _VP - _V))).reshape(_E, _NT)
    dfltt = jnp.pad(dfltt, ((0, 0), (0, 0), (0, _VP - _V))).reshape(_E, _NT)
    primt = jnp.pad(primt, ((0, 0), (0, _NTP - _NT))).reshape(NR, 128)
    dfltt = jnp.pad(dfltt, ((0, 0), (0, _NTP - _NT))).reshape(NR, 128)
    cntp = jnp.pad(counts, ((0, 0), (0, _VP - _V))).reshape(_NT)
    cntp = jnp.pad(cntp, (0, _NTP - _NT)).astype(jnp.float32)
    cntp = jnp.broadcast_to(cntp.reshape(1, _NTP // 128, 128), (_E, _NTP // 128, 128)).reshape(NR, 128)

    blendt, xtp = pl.pallas_call(
        _tc_prep,
        out_shape=(
            jax.ShapeDtypeStruct((NR, 128), jnp.float32),
            jax.ShapeDtypeStruct((32, B), jnp.int32),
        ),
    )(primt, dfltt, cntp, jnp.transpose(X, (1, 0)))

    x4 = xtp.reshape(4, 8, NBT, 128).transpose(0, 2, 1, 3)

    info = plsc.get_sparse_core_info()
    NC, NS = info.num_cores, info.num_subcores
    NW = NC * NS
    assert NW == NBT

    mesh = plsc.VectorSubcoreMesh(core_axis_name="c", subcore_axis_name="s")

    @functools.partial(
        pl.kernel,
        out_type=jax.ShapeDtypeStruct((_F, 8, NBT, 8, 128), jnp.float32),
        mesh=mesh,
        compiler_params=pltpu.CompilerParams(
            use_tc_tiling_on_sc=False,
            needs_layout_passes=False,
            skip_device_barrier=True,
        ),
        scratch_types=[
            pltpu.VMEM((_E * _NTP // 128, 128), jnp.float32),
            pltpu.VMEM((4, 8, 128), jnp.int32),
            pltpu.VMEM((8, 8, 128), jnp.float32),
            pltpu.VMEM((8, 8, 128), jnp.float32),
            pltpu.SemaphoreType.DMA,
            pltpu.SemaphoreType.DMA,
        ],
    )
    def sc_fill(x_hbm, tbl_hbm, out_hbm, tbl_v, idx_v, obuf0, obuf1, sem0, sem1):
        wid = lax.axis_index("s") * NC + lax.axis_index("c")
        pltpu.sync_copy(tbl_hbm, tbl_v)
        pltpu.sync_copy(x_hbm.at[:, wid], idx_v)

        def out_slice(f):
            return out_hbm.at[f, :, wid]

        def chunk(j, obuf, sem):
            ct = j // 4
            ci = (j % 4) * _VP

            xs, ms = [], []
            for c in range(8):
                x16 = idx_v[j // 8, j % 8, pl.ds(c * 16, 16)]
                xs.append(x16 & 15)
                ms.append(x16 < 16)
            lo = tbl_v[ct, pl.ds(ci, 16)]
            hi = tbl_v[ct, pl.ds(ci + 16, 16)]

            @pl.when(j >= 2)
            def _():
                pltpu.make_async_copy(obuf, out_slice(j - 2), sem).wait()
            for e in range(_E):
                if e + 1 < _E:
                    r = 7 * (e + 1) + ct
                    lo_n = tbl_v[r, pl.ds(ci, 16)]
                    hi_n = tbl_v[r, pl.ds(ci + 16, 16)]
                for c in range(8):
                    obuf[e // 8, e % 8, pl.ds(c * 16, 16)] = jnp.where(
                        ms[c], _dg(lo, xs[c]), _dg(hi, xs[c])
                    )
                if e + 1 < _E:
                    lo, hi = lo_n, hi_n
            pltpu.async_copy(obuf, out_slice(j), sem)

        def body(i, carry):
            chunk(2 * i, obuf0, sem0)
            chunk(2 * i + 1, obuf1, sem1)
            return carry

        lax.fori_loop(0, _F // 2, body, 0)
        pltpu.make_async_copy(obuf0, out_slice(_F - 2), sem0).wait()
        pltpu.make_async_copy(obuf1, out_slice(_F - 1), sem1).wait()

    q = sc_fill(x4, blendt)
    return q.transpose((2, 4, 0, 1, 3)).reshape(B, _F, _E)
